# baseline (speedup 1.0000x reference)
import functools

import jax
import jax.numpy as jnp
from jax import lax
from jax.experimental import pallas as pl
from jax.experimental.pallas import tpu as pltpu
from jax.experimental.pallas import tpu_sc as plsc

H = 64
SENT = 200
BATCH = 4096
EPS = 1e-5
NC = 2
NS = 16
NW = NC * NS
SPC = 4
CHUNK = SPC * SENT
SENT_PER_W = BATCH // NW
NCHUNK = SENT_PER_W // SPC

_mesh = plsc.VectorSubcoreMesh(core_axis_name="c", subcore_axis_name="s")


@functools.partial(
    pl.kernel,
    out_type=jax.ShapeDtypeStruct((BATCH, SENT, H), jnp.float32),
    mesh=_mesh,
    scratch_types=[
        pltpu.VMEM((2, SPC, SENT), jnp.int32),
        pltpu.VMEM((2, SPC, SENT, H), jnp.float32),
        pltpu.VMEM((SENT, H), jnp.float32),
        pltpu.VMEM((H,), jnp.float32),
        pltpu.VMEM((H,), jnp.float32),
        pltpu.SemaphoreType.DMA,
        pltpu.SemaphoreType.DMA,
    ],
    compiler_params=pltpu.CompilerParams(
        needs_layout_passes=False, use_tc_tiling_on_sc=False),
)
def _sc_embed_ln(ids_hbm, tok_hbm, pos_hbm, w_hbm, b_hbm, out_hbm,
                 idx_v, rows_v, pos_v, w_v, b_v, gsem, osem):
    cid = lax.axis_index("c")
    sid = lax.axis_index("s")
    wid = sid * NC + cid
    sent_w = wid * SENT_PER_W

    pltpu.sync_copy(pos_hbm.at[pl.ds(0, SENT)], pos_v)
    pltpu.sync_copy(w_hbm, w_v)
    pltpu.sync_copy(b_hbm, b_v)

    def issue(c, b):
        s0 = sent_w + c * SPC
        pltpu.sync_copy(ids_hbm.at[pl.ds(s0, SPC)], idx_v.at[b])
        for s in range(SPC):
            for off, n in ((0, 128), (128, 72)):
                pltpu.async_copy(
                    tok_hbm.at[idx_v.at[b, s, pl.ds(off, n)]],
                    rows_v.at[b, s, pl.ds(off, n)], gsem)

    def drain_gathers(b):
        for s in range(SPC):
            pltpu.make_async_copy(
                tok_hbm.at[idx_v.at[b, s]], rows_v.at[b, s], gsem).wait()

    def drain_out():
        pltpu.make_async_copy(
            rows_v.at[0], out_hbm.at[pl.ds(0, SPC)], osem).wait()

    def compute(b):
        lanes = lax.iota(jnp.int32, 16)
        perms = [lanes ^ m for m in (1, 2, 4, 8)]
        wgt = [w_v[pl.ds(16 * h, 16)] for h in range(4)]
        bia = [b_v[pl.ds(16 * h, 16)] for h in range(4)]

        for s in range(SPC):
            @plsc.parallel_loop(0, SENT, 1, unroll=8)
            def row_loop(r):
                x = []
                for h in range(4):
                    x.append(rows_v[b, s, r, pl.ds(16 * h, 16)]
                             + pos_v[r, pl.ds(16 * h, 16)])
                ss = (x[0] + x[1]) + (x[2] + x[3])
                q = (x[0] * x[0] + x[1] * x[1]) + (x[2] * x[2] + x[3] * x[3])
                for perm in perms:
                    ss = ss + ss.at[perm].get(mode="promise_in_bounds")
                    q = q + q.at[perm].get(mode="promise_in_bounds")
                mv = ss * (1.0 / H)
                vv = q * (1.0 / H) - mv * mv + EPS
                iv = plsc.bitcast(vv, jnp.int32)
                y = plsc.bitcast(
                    jnp.int32(0x5F3759DF) - (iv >> 1), jnp.float32)
                hv = vv * 0.5
                y = y * (1.5 - hv * y * y)
                y = y * (1.5 - hv * y * y)
                my = mv * y
                for h in range(4):
                    rows_v[b, s, r, pl.ds(16 * h, 16)] = (
                        (x[h] * y - my) * wgt[h] + bia[h])

    issue(0, 0)

    @pl.loop(0, NCHUNK // 2)
    def pair_loop(t):
        for b in range(2):
            c = t * 2 + b
            nb = 1 - b

            @pl.when(c + 1 < NCHUNK)
            def _():
                @pl.when(c >= 1)
                def _():
                    drain_out()
                issue(c + 1, nb)

            drain_gathers(b)
            compute(b)
            pltpu.async_copy(
                rows_v.at[b],
                out_hbm.at[pl.ds(sent_w + c * SPC, SPC)], osem)

    drain_out()
    drain_out()


def kernel(batch_token_ids, token_table, pos_table, ln_weight, ln_bias):
    ids = batch_token_ids.astype(jnp.int32)
    return _sc_embed_ln(ids, token_table, pos_table, ln_weight, ln_bias)


# speedup vs baseline: 1.0316x; 1.0316x over previous
import functools

import jax
import jax.numpy as jnp
from jax import lax
from jax.experimental import pallas as pl
from jax.experimental.pallas import tpu as pltpu
from jax.experimental.pallas import tpu_sc as plsc

H = 64
SENT = 200
BATCH = 4096
EPS = 1e-5
NC = 2
NS = 16
NW = NC * NS
SPC = 4
CHUNK = SPC * SENT
SENT_PER_W = BATCH // NW
NCHUNK = SENT_PER_W // SPC

_mesh = plsc.VectorSubcoreMesh(core_axis_name="c", subcore_axis_name="s")


@functools.partial(
    pl.kernel,
    out_type=jax.ShapeDtypeStruct((BATCH, SENT, H), jnp.float32),
    mesh=_mesh,
    scratch_types=[
        pltpu.VMEM((2, SPC, SENT), jnp.int32),
        pltpu.VMEM((2, SPC, SENT, H), jnp.float32),
        pltpu.VMEM((SENT, H), jnp.float32),
        pltpu.VMEM((H,), jnp.float32),
        pltpu.VMEM((H,), jnp.float32),
        pltpu.SemaphoreType.DMA,
        pltpu.SemaphoreType.DMA,
    ],
    compiler_params=pltpu.CompilerParams(
        needs_layout_passes=False, use_tc_tiling_on_sc=False),
)
def _sc_embed_ln(ids_hbm, tok_hbm, pos_hbm, w_hbm, b_hbm, out_hbm,
                 idx_v, rows_v, pos_v, w_v, b_v, gsem, osem):
    cid = lax.axis_index("c")
    sid = lax.axis_index("s")
    wid = sid * NC + cid
    sent_w = wid * SENT_PER_W

    pltpu.sync_copy(pos_hbm.at[pl.ds(0, SENT)], pos_v)
    pltpu.sync_copy(w_hbm, w_v)
    pltpu.sync_copy(b_hbm, b_v)

    def issue(c, b):
        s0 = sent_w + c * SPC
        pltpu.sync_copy(ids_hbm.at[pl.ds(s0, SPC)], idx_v.at[b])
        for s in range(SPC):
            for off, n in ((0, 128), (128, 72)):
                pltpu.async_copy(
                    tok_hbm.at[idx_v.at[b, s, pl.ds(off, n)]],
                    rows_v.at[b, s, pl.ds(off, n)], gsem)

    def drain_gathers(b):
        for s in range(SPC):
            pltpu.make_async_copy(
                tok_hbm.at[idx_v.at[b, s]], rows_v.at[b, s], gsem).wait()

    def drain_out():
        pltpu.make_async_copy(
            rows_v.at[0], out_hbm.at[pl.ds(0, SPC)], osem).wait()

    def compute(b):
        lanes = lax.iota(jnp.int32, 16)
        perms = [lanes ^ m for m in (1, 2, 4, 8)]
        wgt = [w_v[pl.ds(16 * h, 16)] for h in range(4)]
        bia = [b_v[pl.ds(16 * h, 16)] for h in range(4)]

        for s in range(SPC):
            @plsc.parallel_loop(0, SENT, 1, unroll=4)
            def row_loop(r):
                x = []
                for h in range(4):
                    x.append(rows_v[b, s, r, pl.ds(16 * h, 16)]
                             + pos_v[r, pl.ds(16 * h, 16)])
                ss = (x[0] + x[1]) + (x[2] + x[3])
                q = (x[0] * x[0] + x[1] * x[1]) + (x[2] * x[2] + x[3] * x[3])
                for perm in perms:
                    ss = ss + ss.at[perm].get(mode="promise_in_bounds")
                    q = q + q.at[perm].get(mode="promise_in_bounds")
                mv = ss * (1.0 / H)
                vv = q * (1.0 / H) - mv * mv + EPS
                iv = plsc.bitcast(vv, jnp.int32)
                y = plsc.bitcast(
                    jnp.int32(0x5F3759DF) - (iv >> 1), jnp.float32)
                hv = vv * 0.5
                y = y * (1.5 - hv * y * y)
                y = y * (1.5 - hv * y * y)
                my = mv * y
                for h in range(4):
                    rows_v[b, s, r, pl.ds(16 * h, 16)] = (
                        (x[h] * y - my) * wgt[h] + bia[h])

    issue(0, 0)

    @pl.loop(0, NCHUNK // 2)
    def pair_loop(t):
        for b in range(2):
            c = t * 2 + b
            nb = 1 - b

            @pl.when(c + 1 < NCHUNK)
            def _():
                @pl.when(c >= 1)
                def _():
                    drain_out()
                issue(c + 1, nb)

            drain_gathers(b)
            compute(b)
            pltpu.async_copy(
                rows_v.at[b],
                out_hbm.at[pl.ds(sent_w + c * SPC, SPC)], osem)

    drain_out()
    drain_out()


def kernel(batch_token_ids, token_table, pos_table, ln_weight, ln_bias):
    ids = batch_token_ids.astype(jnp.int32)
    return _sc_embed_ln(ids, token_table, pos_table, ln_weight, ln_bias)
